# baseline (speedup 1.0000x reference)
import functools

import jax
import jax.numpy as jnp
from jax import lax
from jax.experimental import pallas as pl
from jax.experimental.pallas import tpu as pltpu
from jax.experimental.pallas import tpu_sc as plsc

SC_PROBLEMS = 12
TC_PB = 2


@functools.lru_cache(maxsize=None)
def _make_sc_kernel(B, S, Q, BSC):
    nQs = BSC * Q
    info = plsc.get_sparse_core_info()
    NC, NS, L = info.num_cores, info.num_subcores, info.num_lanes
    NW = NC * NS
    QW = nQs // NW
    QB = 8
    NBLK = QW // QB
    CH = S // L
    CU = 16
    NITER = NBLK // 2

    mesh = plsc.VectorSubcoreMesh(core_axis_name="c", subcore_axis_name="s")

    @functools.partial(
        pl.kernel,
        out_type=jax.ShapeDtypeStruct((nQs,), jnp.float32),
        mesh=mesh,
        scratch_types=[
            pltpu.VMEM((S,), jnp.float32),
            pltpu.VMEM((QB * S,), jnp.float32),
            pltpu.VMEM((QB * S,), jnp.float32),
            pltpu.VMEM((max(QW, 16),), jnp.float32),
            pltpu.SemaphoreType.DMA,
            pltpu.SemaphoreType.DMA,
        ],
    )
    def sc_kernel(costs_hbm, occ_hbm, out_hbm, costs_v, occ0, occ1, out_v, sem0, sem1):
        wid = lax.axis_index("s") * NC + lax.axis_index("c")
        base_q = wid * QW
        b = base_q // Q
        pltpu.sync_copy(costs_hbm.at[pl.ds(b * S, S)], costs_v)

        def occ_src(blk):
            return occ_hbm.at[pl.ds((base_q + blk * QB) * S, QB * S)]

        pltpu.async_copy(occ_src(0), occ0, sem0)

        lanes = lax.iota(jnp.int32, 16)
        _gdn = lax.GatherDimensionNumbers(
            offset_dims=(), collapsed_slice_dims=(0,), start_index_map=(0,))

        def lane_permute(x, perm):
            return lax.gather(x, perm[:, None], _gdn, slice_sizes=(1,),
                              mode=lax.GatherScatterMode.PROMISE_IN_BOUNDS)

        def lane_allreduce(x):
            for k in (8, 4, 2, 1):
                x = x + lane_permute(x, jnp.bitwise_xor(lanes, k))
            return x

        def compute_block(buf):
            def chunk_body(cc, accs):
                accs = list(accs)
                for u in range(CU):
                    c0 = (cc * CU + u) * L
                    cv = costs_v[pl.ds(c0, L)]
                    for j in range(QB):
                        accs[j] = accs[j] + buf[pl.ds(j * S + c0, L)] * cv
                return tuple(accs)

            init = tuple(jnp.zeros((L,), jnp.float32) for _ in range(QB))
            accs = lax.fori_loop(0, CH // CU, chunk_body, init)
            return [lane_allreduce(a) for a in accs]

        def assemble(sums):
            res = jnp.zeros((16,), jnp.float32)
            for j, s in enumerate(sums):
                res = jnp.where(lanes == j, s, res)
            return res

        if NITER >= 1:
            def body(i, carry):
                blk0 = 2 * i
                blk1 = 2 * i + 1
                pltpu.async_copy(occ_src(blk1), occ1, sem1)
                pltpu.make_async_copy(occ_src(blk0), occ0, sem0).wait()
                sums0 = compute_block(occ0)

                @pl.when(i < NITER - 1)
                def _():
                    pltpu.async_copy(occ_src(blk0 + 2), occ0, sem0)

                pltpu.make_async_copy(occ_src(blk1), occ1, sem1).wait()
                sums1 = compute_block(occ1)

                out_v[pl.ds(i * 16, 16)] = assemble(sums0 + sums1)
                return carry

            lax.fori_loop(0, NITER, body, 0)
            pltpu.sync_copy(out_v, out_hbm.at[pl.ds(base_q, QW)])
        else:
            pltpu.make_async_copy(occ_src(0), occ0, sem0).wait()
            out_v[pl.ds(0, 16)] = assemble(compute_block(occ0))
            pltpu.sync_copy(out_v.at[pl.ds(0, QW)], out_hbm.at[pl.ds(base_q, QW)])

    return sc_kernel


def _tc_body(costs_ref, occ_ref, out_ref):
    nR, SW, _ = occ_ref.shape
    Q = nR // TC_PB
    for p in range(TC_PB):
        acc = jnp.zeros((Q, 1), jnp.float32)
        for t in range(SW):
            acc = acc + lax.dot_general(
                occ_ref[p * Q:(p + 1) * Q, t, :], costs_ref[p, t:t + 1, :],
                dimension_numbers=(((1,), (1,)), ((), ())),
                preferred_element_type=jnp.float32)
        out_ref[p * Q:(p + 1) * Q, :] = acc


def kernel(costs_flat, occ_flat, valid, costs_row_splits, question_row_splits, occ_inner_splits):
    B = valid.shape[0]
    nQ = occ_inner_splits.shape[0] - 1
    S = costs_flat.shape[0] // B
    Q = nQ // B
    SW = S // 128
    BSC = SC_PROBLEMS
    BTC = B - BSC

    sc = _make_sc_kernel(B, S, Q, BSC)
    sc_logits = sc(costs_flat, occ_flat)

    occ3 = occ_flat.reshape(nQ, SW, 128)
    costs3 = costs_flat.reshape(B, SW, 128)
    tc_out = pl.pallas_call(
        _tc_body,
        grid=(BTC // TC_PB,),
        in_specs=[
            pl.BlockSpec((TC_PB, SW, 128), lambda i: (i + BSC // TC_PB, 0, 0)),
            pl.BlockSpec((TC_PB * Q, SW, 128), lambda i: (i + BSC // TC_PB, 0, 0)),
        ],
        out_specs=pl.BlockSpec((TC_PB * Q, 1), lambda i: (i, 0)),
        out_shape=jax.ShapeDtypeStruct((BTC * Q, 1), jnp.float32),
    )(costs3, occ3)

    logits = jnp.concatenate([sc_logits, tc_out.reshape(BTC * Q)])
    q_valid = jnp.broadcast_to(valid[:, None], (B, Q)).reshape(nQ)
    return jnp.where(q_valid, logits, 0.0)


# speedup vs baseline: 1.0365x; 1.0365x over previous
import functools

import jax
import jax.numpy as jnp
from jax import lax
from jax.experimental import pallas as pl
from jax.experimental.pallas import tpu as pltpu
from jax.experimental.pallas import tpu_sc as plsc

SC_PROBLEMS = 8
TC_PB = 2


@functools.lru_cache(maxsize=None)
def _make_sc_kernel(B, S, Q, BSC):
    nQs = BSC * Q
    info = plsc.get_sparse_core_info()
    NC, NS, L = info.num_cores, info.num_subcores, info.num_lanes
    NW = NC * NS
    QW = nQs // NW
    QB = 8
    NBLK = QW // QB
    CH = S // L
    CU = 16
    NITER = NBLK // 2

    mesh = plsc.VectorSubcoreMesh(core_axis_name="c", subcore_axis_name="s")

    @functools.partial(
        pl.kernel,
        out_type=jax.ShapeDtypeStruct((nQs,), jnp.float32),
        mesh=mesh,
        scratch_types=[
            pltpu.VMEM((S,), jnp.float32),
            pltpu.VMEM((QB * S,), jnp.float32),
            pltpu.VMEM((QB * S,), jnp.float32),
            pltpu.VMEM((max(QW, 16),), jnp.float32),
            pltpu.SemaphoreType.DMA,
            pltpu.SemaphoreType.DMA,
        ],
    )
    def sc_kernel(costs_hbm, occ_hbm, out_hbm, costs_v, occ0, occ1, out_v, sem0, sem1):
        wid = lax.axis_index("s") * NC + lax.axis_index("c")
        base_q = wid * QW
        b = base_q // Q
        pltpu.sync_copy(costs_hbm.at[pl.ds(b * S, S)], costs_v)

        def occ_src(blk):
            return occ_hbm.at[pl.ds((base_q + blk * QB) * S, QB * S)]

        pltpu.async_copy(occ_src(0), occ0, sem0)

        lanes = lax.iota(jnp.int32, 16)
        _gdn = lax.GatherDimensionNumbers(
            offset_dims=(), collapsed_slice_dims=(0,), start_index_map=(0,))

        def lane_permute(x, perm):
            return lax.gather(x, perm[:, None], _gdn, slice_sizes=(1,),
                              mode=lax.GatherScatterMode.PROMISE_IN_BOUNDS)

        def lane_allreduce(x):
            for k in (8, 4, 2, 1):
                x = x + lane_permute(x, jnp.bitwise_xor(lanes, k))
            return x

        def compute_block(buf):
            def chunk_body(cc, accs):
                accs = list(accs)
                for u in range(CU):
                    c0 = (cc * CU + u) * L
                    cv = costs_v[pl.ds(c0, L)]
                    for j in range(QB):
                        accs[j] = accs[j] + buf[pl.ds(j * S + c0, L)] * cv
                return tuple(accs)

            init = tuple(jnp.zeros((L,), jnp.float32) for _ in range(QB))
            accs = lax.fori_loop(0, CH // CU, chunk_body, init)
            return [lane_allreduce(a) for a in accs]

        def assemble(sums):
            res = jnp.zeros((16,), jnp.float32)
            for j, s in enumerate(sums):
                res = jnp.where(lanes == j, s, res)
            return res

        if NITER >= 1:
            def body(i, carry):
                blk0 = 2 * i
                blk1 = 2 * i + 1
                pltpu.async_copy(occ_src(blk1), occ1, sem1)
                pltpu.make_async_copy(occ_src(blk0), occ0, sem0).wait()
                sums0 = compute_block(occ0)

                @pl.when(i < NITER - 1)
                def _():
                    pltpu.async_copy(occ_src(blk0 + 2), occ0, sem0)

                pltpu.make_async_copy(occ_src(blk1), occ1, sem1).wait()
                sums1 = compute_block(occ1)

                out_v[pl.ds(i * 16, 16)] = assemble(sums0 + sums1)
                return carry

            lax.fori_loop(0, NITER, body, 0)
            pltpu.sync_copy(out_v, out_hbm.at[pl.ds(base_q, QW)])
        else:
            pltpu.make_async_copy(occ_src(0), occ0, sem0).wait()
            out_v[pl.ds(0, 16)] = assemble(compute_block(occ0))
            pltpu.sync_copy(out_v.at[pl.ds(0, QW)], out_hbm.at[pl.ds(base_q, QW)])

    return sc_kernel


def _tc_body(costs_ref, occ_ref, out_ref):
    nR, SW, _ = occ_ref.shape
    Q = nR // TC_PB
    for p in range(TC_PB):
        acc = jnp.zeros((Q, 1), jnp.float32)
        for t in range(SW):
            acc = acc + lax.dot_general(
                occ_ref[p * Q:(p + 1) * Q, t, :], costs_ref[p, t:t + 1, :],
                dimension_numbers=(((1,), (1,)), ((), ())),
                preferred_element_type=jnp.float32)
        out_ref[p * Q:(p + 1) * Q, :] = acc


def kernel(costs_flat, occ_flat, valid, costs_row_splits, question_row_splits, occ_inner_splits):
    B = valid.shape[0]
    nQ = occ_inner_splits.shape[0] - 1
    S = costs_flat.shape[0] // B
    Q = nQ // B
    SW = S // 128
    BSC = SC_PROBLEMS
    BTC = B - BSC

    costs_m = (costs_flat.reshape(B, S)
               * valid.astype(jnp.float32)[:, None]).reshape(B * S)

    sc = _make_sc_kernel(B, S, Q, BSC)
    sc_logits = sc(costs_m, occ_flat)

    occ3 = occ_flat.reshape(nQ, SW, 128)
    costs3 = costs_m.reshape(B, SW, 128)
    tc_out = pl.pallas_call(
        _tc_body,
        grid=(BTC // TC_PB,),
        in_specs=[
            pl.BlockSpec((TC_PB, SW, 128), lambda i: (i + BSC // TC_PB, 0, 0)),
            pl.BlockSpec((TC_PB * Q, SW, 128), lambda i: (i + BSC // TC_PB, 0, 0)),
        ],
        out_specs=pl.BlockSpec((TC_PB * Q, 1), lambda i: (i, 0)),
        out_shape=jax.ShapeDtypeStruct((BTC * Q, 1), jnp.float32),
    )(costs3, occ3)

    return jnp.concatenate([sc_logits, tc_out.reshape(BTC * Q)])


# speedup vs baseline: 1.0560x; 1.0189x over previous
import functools

import jax
import jax.numpy as jnp
from jax import lax
from jax.experimental import pallas as pl
from jax.experimental.pallas import tpu as pltpu
from jax.experimental.pallas import tpu_sc as plsc

SC_PROBLEMS = 8
TC_PB = 2


@functools.lru_cache(maxsize=None)
def _make_sc_kernel(B, S, Q, BSC):
    nQs = BSC * Q
    info = plsc.get_sparse_core_info()
    NC, NS, L = info.num_cores, info.num_subcores, info.num_lanes
    NW = NC * NS
    QW = nQs // NW
    QB = 8
    NBLK = QW // QB
    CH = S // L
    CU = 16
    NITER = NBLK // 2

    mesh = plsc.VectorSubcoreMesh(core_axis_name="c", subcore_axis_name="s")

    @functools.partial(
        pl.kernel,
        out_type=jax.ShapeDtypeStruct((nQs,), jnp.float32),
        mesh=mesh,
        scratch_types=[
            pltpu.VMEM((S,), jnp.float32),
            pltpu.VMEM((QB * S,), jnp.float32),
            pltpu.VMEM((QB * S,), jnp.float32),
            pltpu.VMEM((max(QW, 16),), jnp.float32),
            pltpu.SemaphoreType.DMA,
            pltpu.SemaphoreType.DMA,
        ],
    )
    def sc_kernel(costs_hbm, occ_hbm, out_hbm, costs_v, occ0, occ1, out_v, sem0, sem1):
        wid = lax.axis_index("s") * NC + lax.axis_index("c")
        base_q = wid * QW
        b = base_q // Q
        pltpu.sync_copy(costs_hbm.at[pl.ds(b * S, S)], costs_v)

        def occ_src(blk):
            return occ_hbm.at[pl.ds((base_q + blk * QB) * S, QB * S)]

        pltpu.async_copy(occ_src(0), occ0, sem0)

        lanes = lax.iota(jnp.int32, 16)
        _gdn = lax.GatherDimensionNumbers(
            offset_dims=(), collapsed_slice_dims=(0,), start_index_map=(0,))

        def lane_permute(x, perm):
            return lax.gather(x, perm[:, None], _gdn, slice_sizes=(1,),
                              mode=lax.GatherScatterMode.PROMISE_IN_BOUNDS)

        def lane_allreduce(x):
            for k in (8, 4, 2, 1):
                x = x + lane_permute(x, jnp.bitwise_xor(lanes, k))
            return x

        def compute_block(buf):
            def chunk_body(cc, accs):
                accs = list(accs)
                for u in range(CU):
                    c0 = (cc * CU + u) * L
                    cv = costs_v[pl.ds(c0, L)]
                    for j in range(QB):
                        accs[j] = accs[j] + buf[pl.ds(j * S + c0, L)] * cv
                return tuple(accs)

            init = tuple(jnp.zeros((L,), jnp.float32) for _ in range(QB))
            accs = lax.fori_loop(0, CH // CU, chunk_body, init)
            return [lane_allreduce(a) for a in accs]

        def assemble(sums):
            res = jnp.zeros((16,), jnp.float32)
            for j, s in enumerate(sums):
                res = jnp.where(lanes == j, s, res)
            return res

        if NITER >= 1:
            def body(i, carry):
                blk0 = 2 * i
                blk1 = 2 * i + 1
                pltpu.async_copy(occ_src(blk1), occ1, sem1)
                pltpu.make_async_copy(occ_src(blk0), occ0, sem0).wait()
                sums0 = compute_block(occ0)

                @pl.when(i < NITER - 1)
                def _():
                    pltpu.async_copy(occ_src(blk0 + 2), occ0, sem0)

                pltpu.make_async_copy(occ_src(blk1), occ1, sem1).wait()
                sums1 = compute_block(occ1)

                out_v[pl.ds(i * 16, 16)] = assemble(sums0 + sums1)
                return carry

            lax.fori_loop(0, NITER, body, 0)
            pltpu.sync_copy(out_v, out_hbm.at[pl.ds(base_q, QW)])
        else:
            pltpu.make_async_copy(occ_src(0), occ0, sem0).wait()
            out_v[pl.ds(0, 16)] = assemble(compute_block(occ0))
            pltpu.sync_copy(out_v.at[pl.ds(0, QW)], out_hbm.at[pl.ds(base_q, QW)])

    return sc_kernel


def _tc_body(costs_ref, occ_ref, out_ref):
    nR, SW, _ = occ_ref.shape
    Q = nR // TC_PB
    for p in range(TC_PB):
        acc = jnp.zeros((Q, 1), jnp.float32)
        for t in range(SW):
            acc = acc + lax.dot_general(
                occ_ref[p * Q:(p + 1) * Q, t, :], costs_ref[p, t:t + 1, :],
                dimension_numbers=(((1,), (1,)), ((), ())),
                preferred_element_type=jnp.float32)
        out_ref[p * Q:(p + 1) * Q, :] = acc


def kernel(costs_flat, occ_flat, valid, costs_row_splits, question_row_splits, occ_inner_splits):
    B = valid.shape[0]
    nQ = occ_inner_splits.shape[0] - 1
    S = costs_flat.shape[0] // B
    Q = nQ // B
    SW = S // 128
    BSC = SC_PROBLEMS
    BTC = B - BSC

    sc = _make_sc_kernel(B, S, Q, BSC)
    sc_logits = sc(costs_flat, occ_flat)

    occ3 = occ_flat.reshape(nQ, SW, 128)
    costs3 = costs_flat.reshape(B, SW, 128)
    tc_out = pl.pallas_call(
        _tc_body,
        grid=(BTC // TC_PB,),
        in_specs=[
            pl.BlockSpec((TC_PB, SW, 128), lambda i: (i + BSC // TC_PB, 0, 0)),
            pl.BlockSpec((TC_PB * Q, SW, 128), lambda i: (i + BSC // TC_PB, 0, 0)),
        ],
        out_specs=pl.BlockSpec((TC_PB * Q, 1), lambda i: (i, 0)),
        out_shape=jax.ShapeDtypeStruct((BTC * Q, 1), jnp.float32),
    )(costs3, occ3)

    logits = jnp.concatenate([sc_logits, tc_out.reshape(BTC * Q)])
    q_valid = jnp.broadcast_to(valid[:, None], (B, Q)).reshape(nQ)
    return jnp.where(q_valid, logits, 0.0)
